# baseline (speedup 1.0000x reference)
import functools

import jax
import jax.numpy as jnp
from jax import lax
from jax.experimental import pallas as pl
from jax.experimental.pallas import tpu as pltpu
from jax.experimental.pallas import tpu_sc as plsc

_NC = 2
_NS = 16
_NW = _NC * _NS
_CH = 128
_GC = 2
_GE = _GC * _CH
_NB = 2


def _vp_matmul_kernel(v_ref, w_ref, o_ref):
    o_ref[...] = jnp.dot(v_ref[...], w_ref[...],
                         preferred_element_type=jnp.float32).astype(jnp.bfloat16)


def _vp_matmul(v, w):
    BN, d = v.shape
    blk = 1000
    return pl.pallas_call(
        _vp_matmul_kernel,
        grid=(BN // blk,),
        in_specs=[pl.BlockSpec((blk, d), lambda i: (i, 0)),
                  pl.BlockSpec((d, d), lambda i: (0, 0))],
        out_specs=pl.BlockSpec((blk, d), lambda i: (i, 0)),
        out_shape=jax.ShapeDtypeStruct((BN, d), jnp.bfloat16),
    )(v, w)


def _make_sc_gather(Q, W0, W1, gtot, d, de):
    mesh = plsc.VectorSubcoreMesh(core_axis_name="c", subcore_axis_name="s")
    wmax = max(W0, W1)
    vrows = _GE // 2
    erows = _GE * de // 128

    scratch = [pltpu.VMEM((wmax * _GC, _CH), jnp.int32),
               pltpu.VMEM((wmax * _GC, _CH), jnp.int32)]
    for _ in range(_NB):
        scratch.append(pltpu.VMEM((_GE, d), jnp.bfloat16))
    for _ in range(_NB):
        scratch.append(pltpu.VMEM((vrows, 128), jnp.float32))
    for _ in range(_NB):
        scratch.append(pltpu.VMEM((_GE, de), jnp.float32))
    for _ in range(_NB):
        scratch.append(pltpu.VMEM((erows, 128), jnp.float32))
    for _ in range(2 * _NB):
        scratch.append(pltpu.SemaphoreType.DMA)

    @functools.partial(
        pl.kernel, mesh=mesh,
        out_type=(jax.ShapeDtypeStruct((gtot, vrows, 128), jnp.float32),
                  jax.ShapeDtypeStruct((gtot, erows, 128), jnp.float32)),
        scratch_types=scratch,
        compiler_params=pltpu.CompilerParams(use_tc_tiling_on_sc=False,
                                             needs_layout_passes=False),
    )
    def gather_k(vp_hbm, aidx_hbm, et_hbm, bidx_hbm, vg_hbm, eg_hbm, *scr):
        aiv, biv = scr[0], scr[1]
        vb16 = scr[2:2 + _NB]
        vbp = scr[2 + _NB:2 + 2 * _NB]
        ebuf = scr[2 + 2 * _NB:2 + 3 * _NB]
        ebp = scr[2 + 3 * _NB:2 + 4 * _NB]
        gsems = scr[2 + 4 * _NB:2 + 5 * _NB]
        wsems = scr[2 + 5 * _NB:2 + 6 * _NB]

        c = lax.axis_index("c")
        s = lax.axis_index("s")
        start = Q * s + W0 * c
        count = W0 + (W1 - W0) * c
        pltpu.sync_copy(aidx_hbm.at[pl.ds(start * _GC, wmax * _GC)], aiv)
        pltpu.sync_copy(bidx_hbm.at[pl.ds(start * _GC, wmax * _GC)], biv)

        def fire_g(k, b):
            for j in range(_GC):
                pltpu.async_copy(vp_hbm.at[aiv.at[k * _GC + j]],
                                 vb16[b].at[pl.ds(j * _CH, _CH)], gsems[b])
                pltpu.async_copy(et_hbm.at[biv.at[k * _GC + j]],
                                 ebuf[b].at[pl.ds(j * _CH, _CH)], gsems[b])

        def drain_g(k, b):
            for j in range(_GC):
                pltpu.make_async_copy(vp_hbm.at[aiv.at[k * _GC + j]],
                                      vb16[b].at[pl.ds(j * _CH, _CH)],
                                      gsems[b]).wait()
                pltpu.make_async_copy(et_hbm.at[biv.at[k * _GC + j]],
                                      ebuf[b].at[pl.ds(j * _CH, _CH)],
                                      gsems[b]).wait()

        def repack(b):
            def vrow(r):
                for i in range(4):
                    vbp[b][r, pl.ds(16 * i, 16)] = plsc.bitcast(
                        vb16[b][2 * r, pl.ds(32 * i, 32)], jnp.float32)
                    vbp[b][r, pl.ds(64 + 16 * i, 16)] = plsc.bitcast(
                        vb16[b][2 * r + 1, pl.ds(32 * i, 32)], jnp.float32)
            pl.loop(0, vrows)(vrow)
            def erow(q):
                for j in range(8):
                    ebp[b][q, pl.ds(16 * j, 16)] = ebuf[b][8 * q + j, :]
            pl.loop(0, erows)(erow)

        def fire_wb(k, b):
            g = start + k
            pltpu.async_copy(vbp[b], vg_hbm.at[g], wsems[b])
            pltpu.async_copy(ebp[b], eg_hbm.at[g], wsems[b])

        def wait_wb(k, b):
            g = start + k
            pltpu.make_async_copy(vbp[b], vg_hbm.at[g], wsems[b]).wait()
            pltpu.make_async_copy(ebp[b], eg_hbm.at[g], wsems[b]).wait()

        for b in range(_NB):
            fire_g(b, b)

        def body(g):
            for b in range(_NB):
                k = g + b
                drain_g(k, b)
                repack(b)
                fire_wb(k, b)
                wait_wb(k, b)
                fire_g(k + _NB, b)

        pl.loop(0, count - _NB, step=_NB)(body)

        for b in range(_NB):
            k = count - _NB + b
            drain_g(k, b)
            repack(b)
            fire_wb(k, b)
            wait_wb(k, b)

    return gather_k


def _gelu(x):
    return 0.5 * x * (1.0 + lax.erf(x * 0.7071067811865476))


def _fused_kernel(vgp_ref, egp_ref, v_ref, h0_ref, nme_ref, nmo_ref,
                  we_ref, b1_ref, w2a_ref, w2b_ref, b2_ref, wf_ref, bf_ref,
                  o_ref):
    K, d = v_ref.shape
    nh = nme_ref.shape[-1]
    h = d // 2
    x16 = pltpu.bitcast(vgp_ref[...], jnp.bfloat16)
    z = x16.reshape(K * nh, 2, d)
    zl = z[:, 0, :]
    zh = z[:, 1, :]
    a0 = jnp.concatenate([zl[:, :h], zh[:, :h]],
                         axis=1).astype(jnp.float32)
    a1 = jnp.concatenate([zl[:, h:], zh[:, h:]],
                         axis=1).astype(jnp.float32)
    egp = egp_ref[...]
    we = we_ref[...]
    eps = [jnp.dot(egp[:, 16 * j:16 * (j + 1)], we,
                   preferred_element_type=jnp.float32) for j in range(8)]
    ep_e = jnp.stack([eps[0], eps[2], eps[4], eps[6]], axis=1).reshape(K * nh, d)
    ep_o = jnp.stack([eps[1], eps[3], eps[5], eps[7]], axis=1).reshape(K * nh, d)
    b1 = b1_ref[...]
    ye = _gelu(a0 + ep_e + b1).reshape(K, nh, d) * nme_ref[...][:, :, None]
    yo = _gelu(a1 + ep_o + b1).reshape(K, nh, d) * nmo_ref[...][:, :, None]
    nl = jnp.sum(ye, axis=1) + jnp.sum(yo, axis=1)
    sup = (jnp.dot(nl, w2a_ref[...], preferred_element_type=jnp.float32)
           + jnp.dot(v_ref[...], w2b_ref[...], preferred_element_type=jnp.float32)
           + b2_ref[...] + h0_ref[...])
    o_ref[...] = (jnp.dot(sup, wf_ref[...], preferred_element_type=jnp.float32)
                  + bf_ref[...])


def _fused(vgp, egp, v, h0s, nme, nmo, we, b1, w2a, w2b, b2, wf, bfu):
    BN, d = v.shape
    nh = nme.shape[-1]
    K = 200
    KV = K * nh
    KEg = K * 2 * nh * 16 // 128
    grid = (BN // K,)
    full = lambda i: (0, 0)
    return pl.pallas_call(
        _fused_kernel,
        grid=grid,
        in_specs=[
            pl.BlockSpec((KV, 128), lambda i: (i, 0)),
            pl.BlockSpec((KEg, 128), lambda i: (i, 0)),
            pl.BlockSpec((K, d), lambda i: (i, 0)),
            pl.BlockSpec((K, d), lambda i: (i, 0)),
            pl.BlockSpec((K, nh), lambda i: (i, 0)),
            pl.BlockSpec((K, nh), lambda i: (i, 0)),
            pl.BlockSpec((16, d), full),
            pl.BlockSpec((1, d), full),
            pl.BlockSpec((d, d), full),
            pl.BlockSpec((d, d), full),
            pl.BlockSpec((1, d), full),
            pl.BlockSpec((d, d), full),
            pl.BlockSpec((1, d), full),
        ],
        out_specs=pl.BlockSpec((K, d), lambda i: (i, 0)),
        out_shape=jax.ShapeDtypeStruct((BN, d), jnp.float32),
        compiler_params=pltpu.CompilerParams(
            dimension_semantics=("parallel",)),
    )(vgp, egp, v, h0s, nme, nmo, we, b1, w2a, w2b, b2, wf, bfu)


def kernel(vertex_features, atom_adj, bond_adj, h0, lamda, alpha, l,
           edge_initial, vertex_mask, nbs_mask,
           W_u1, b_u1, W_u2, b_u2, W_fu, b_fu):
    B, N = vertex_mask.shape
    n_nbs = nbs_mask.shape[2]
    d = vertex_features.shape[-1]
    bf = edge_initial.shape[-1]
    BN = B * N
    E = atom_adj.shape[0]

    V = vertex_features.reshape(BN, d)

    theta = jnp.asarray(jnp.log(lamda / l + 1), jnp.float32)
    one_m_a = jnp.asarray(1.0 - alpha, jnp.float32)
    a_f = jnp.asarray(alpha, jnp.float32)
    W2a = W_u2[:d] * one_m_a
    W2b = W_u2[d:] * one_m_a
    b2 = (b_u2 * one_m_a).reshape(1, d)
    Wf_eff = theta * W_fu + (1.0 - theta) * jnp.eye(d, dtype=jnp.float32)
    bf_eff = (theta * b_fu).reshape(1, d)
    h0s = (a_f * h0).reshape(BN, d)
    b1 = b_u1.reshape(1, d)

    vp16 = _vp_matmul(V, W_u1[:d])

    gtot_raw = -(-E // _GE)
    Q = -(-gtot_raw // _NS)
    Q += Q % 2
    gtot = Q * _NS
    W0 = int(round(Q * 0.775 / 2)) * 2
    W1 = Q - W0
    E_pad = gtot * _GE
    pad = E_pad - E
    aidx = jnp.pad(atom_adj, (0, pad)).reshape(gtot * _GC, _CH)
    bidx = jnp.pad(bond_adj, (0, pad)).reshape(gtot * _GC, _CH)
    vgp, egp = _make_sc_gather(Q, W0, W1, gtot, d, bf)(
        vp16, aidx, edge_initial, bidx)

    vgp = vgp.reshape(E_pad // 2, 128)
    egp = egp.reshape(E_pad * bf // 128, 128)

    perm = jnp.concatenate([jnp.arange(0, d, 2), jnp.arange(1, d, 2)])
    we_p = W_u1[d:][:, perm]
    b1_p = b_u1[perm].reshape(1, d)
    W2a_p = W2a[perm, :]
    nm = nbs_mask.reshape(BN, n_nbs)
    nme = nm[:, 0::2]
    nmo = nm[:, 1::2]
    out = _fused(vgp, egp, V, h0s, nme, nmo, we_p, b1_p, W2a_p, W2b, b2,
                 Wf_eff, bf_eff)
    return out.reshape(B, N, d)
